# baseline (speedup 1.0000x reference)
import functools

import jax
import jax.numpy as jnp
from jax import lax
from jax.experimental import pallas as pl
from jax.experimental.pallas import tpu as pltpu, tpu_sc as plsc

N = 10000
E = 320000
D = 128
DH = D // 2

NC = 2
NS = 16
CHUNK = 128
NCH = 160
E_PAD = NS * NCH * CHUNK
SINK = N
R = 10112
RPT = R // NS
BLK = 2000
GRID = N // BLK
CW = 8
NB = 4


def _sc_scatter(with_counts):
    out_type = [jax.ShapeDtypeStruct((NC, R, DH), jnp.float32)]
    if with_counts:
        out_type.append(jax.ShapeDtypeStruct((NC, R, CW), jnp.float32))
    scratch = [
        pltpu.VMEM((NCH, CHUNK), jnp.int32),
        pltpu.VMEM((NCH, CHUNK), jnp.int32),
        pltpu.VMEM((NB, CHUNK, DH), jnp.float32),
        pltpu.VMEM((CHUNK, CW), jnp.float32),
        pltpu.VMEM_SHARED((R, DH), jnp.float32),
        pltpu.VMEM_SHARED((R, CW), jnp.float32),
    ] + [pltpu.SemaphoreType.DMA] * NB

    mesh = plsc.VectorSubcoreMesh(core_axis_name="c", subcore_axis_name="s")

    @functools.partial(
        pl.kernel, out_type=tuple(out_type), mesh=mesh,
        scratch_types=scratch,
        compiler_params=pltpu.CompilerParams(use_tc_tiling_on_sc=False))
    def k(x_hbm, src_hbm, dst_hbm, zeros_hbm, zeros_c_hbm, ones_hbm,
          part_hbm, *rest):
        if with_counts:
            cnt_hbm = rest[0]
            rest = rest[1:]
        src_v, dst_v, rows_v, ones_v, acc, cacc = rest[:6]
        sems_g = rest[6:]
        c = lax.axis_index("c")
        s = lax.axis_index("s")

        pltpu.sync_copy(zeros_hbm, acc.at[pl.ds(s * RPT, RPT)])
        if with_counts:
            pltpu.sync_copy(zeros_c_hbm, cacc.at[pl.ds(s * RPT, RPT)])
            pltpu.sync_copy(ones_hbm, ones_v)
        pltpu.sync_copy(src_hbm.at[s], src_v)
        pltpu.sync_copy(dst_hbm.at[s], dst_v)
        plsc.subcore_barrier()

        zc = x_hbm.at[c]

        for b in range(NB):
            pltpu.async_copy(zc.at[src_v.at[b]], rows_v.at[b], sems_g[b])

        @pl.loop(0, NCH, step=NB)
        def chunk_loop(j):
            for b in range(NB):
                jj = j + b
                pltpu.make_async_copy(
                    zc.at[src_v.at[jj]], rows_v.at[b], sems_g[b]).wait()
                pltpu.sync_copy(rows_v.at[b], acc.at[dst_v.at[jj]], add=True)
                if with_counts:
                    @pl.when(c == b % NC)
                    def _():
                        pltpu.sync_copy(ones_v, cacc.at[dst_v.at[jj]],
                                        add=True)
                nxt = jj + NB

                @pl.when(nxt < NCH)
                def _():
                    pltpu.async_copy(zc.at[src_v.at[nxt]], rows_v.at[b],
                                     sems_g[b])

        plsc.subcore_barrier()
        rows = pl.ds(s * RPT, RPT)
        pltpu.sync_copy(acc.at[rows], part_hbm.at[c, rows])
        if with_counts:
            pltpu.sync_copy(cacc.at[rows], cnt_hbm.at[c, rows])

    return k


_sc_scatter_l1 = _sc_scatter(True)
_sc_scatter_l2 = _sc_scatter(False)


def _dotT(a, w):
    return lax.dot_general(a, w, (((1,), (1,)), ((), ())),
                           preferred_element_type=jnp.float32)


def _tc_root(x, Wr, b):
    def body(x_ref, wr_ref, b_ref, y_ref):
        y_ref[...] = _dotT(x_ref[...], wr_ref[...]) + b_ref[...]

    return pl.pallas_call(
        body,
        grid=(GRID,),
        in_specs=[
            pl.BlockSpec((BLK, D), lambda i: (i, 0)),
            pl.BlockSpec((D, D), lambda i: (0, 0)),
            pl.BlockSpec((1, D), lambda i: (0, 0)),
        ],
        out_specs=pl.BlockSpec((BLK, D), lambda i: (i, 0)),
        out_shape=jax.ShapeDtypeStruct((N, D), jnp.float32),
    )(x, Wr, b.reshape(1, D))


def _tc_mid(p, cnt, y1, Wl, Wr, b):
    def body(p_ref, c_ref, y1_ref, wl_ref, wr_ref, b_ref, h_ref, y_ref):
        agg = jnp.concatenate([p_ref[0], p_ref[1]], axis=1)
        cn = c_ref[0, :, :1] + c_ref[1, :, :1]
        mean = agg * (1.0 / jnp.maximum(cn, 1.0))
        h = jnp.maximum(_dotT(mean, wl_ref[...]) + y1_ref[...], 0.0)
        h_ref[0] = h[:, :DH]
        h_ref[1] = h[:, DH:]
        y_ref[...] = _dotT(h, wr_ref[...]) + b_ref[...]

    h, y2 = pl.pallas_call(
        body,
        grid=(GRID,),
        in_specs=[
            pl.BlockSpec((NC, BLK, DH), lambda i: (0, i, 0)),
            pl.BlockSpec((NC, BLK, CW), lambda i: (0, i, 0)),
            pl.BlockSpec((BLK, D), lambda i: (i, 0)),
            pl.BlockSpec((D, D), lambda i: (0, 0)),
            pl.BlockSpec((D, D), lambda i: (0, 0)),
            pl.BlockSpec((1, D), lambda i: (0, 0)),
        ],
        out_specs=[
            pl.BlockSpec((NC, BLK, DH), lambda i: (0, i, 0)),
            pl.BlockSpec((BLK, D), lambda i: (i, 0)),
        ],
        out_shape=[
            jax.ShapeDtypeStruct((NC, N, DH), jnp.float32),
            jax.ShapeDtypeStruct((N, D), jnp.float32),
        ],
    )(p, cnt, y1, Wl, Wr, b.reshape(1, D))
    return h, y2


def _tc_out(q, cnt, y2, Wl):
    def body(q_ref, c_ref, y2_ref, wl_ref, o_ref):
        agg = jnp.concatenate([q_ref[0], q_ref[1]], axis=1)
        cn = c_ref[0, :, :1] + c_ref[1, :, :1]
        mean = agg * (1.0 / jnp.maximum(cn, 1.0))
        o_ref[...] = _dotT(mean, wl_ref[...]) + y2_ref[...]

    return pl.pallas_call(
        body,
        grid=(GRID,),
        in_specs=[
            pl.BlockSpec((NC, BLK, DH), lambda i: (0, i, 0)),
            pl.BlockSpec((NC, BLK, CW), lambda i: (0, i, 0)),
            pl.BlockSpec((BLK, D), lambda i: (i, 0)),
            pl.BlockSpec((D, D), lambda i: (0, 0)),
        ],
        out_specs=pl.BlockSpec((BLK, D), lambda i: (i, 0)),
        out_shape=jax.ShapeDtypeStruct((N, D), jnp.float32),
    )(q, cnt, y2, Wl)


def kernel(x, edge_index, W1l, b1l, W1r, W2l, b2l, W2r):
    src = edge_index[0]
    dst = edge_index[1]
    pad = E_PAD - E
    pad_src = (jnp.arange(pad, dtype=src.dtype) * 13) % N
    pad_dst = SINK + (jnp.arange(pad, dtype=dst.dtype) % (R - SINK))
    src_p = jnp.concatenate([src, pad_src]
                            ).reshape(NS, NCH, CHUNK).astype(jnp.int32)
    dst_p = jnp.concatenate([dst, pad_dst]
                            ).reshape(NS, NCH, CHUNK).astype(jnp.int32)
    zeros = jnp.zeros((RPT, DH), jnp.float32)
    zeros_c = jnp.zeros((RPT, CW), jnp.float32)
    ones = jnp.ones((CHUNK, CW), jnp.float32)

    xs = jnp.stack([x[:, :DH], x[:, DH:]])
    p, cnt = _sc_scatter_l1(xs, src_p, dst_p, zeros, zeros_c, ones)
    y1 = _tc_root(x, W1r, b1l)
    hs, y2 = _tc_mid(p, cnt, y1, W1l, W2r, b2l)
    (q,) = _sc_scatter_l2(hs, src_p, dst_p, zeros, zeros_c, ones)
    return _tc_out(q, cnt, y2, W2l)


# speedup vs baseline: 1.0817x; 1.0817x over previous
import functools

import jax
import jax.numpy as jnp
from jax import lax
from jax.experimental import pallas as pl
from jax.experimental.pallas import tpu as pltpu, tpu_sc as plsc

N = 10000
E = 320000
D = 128
DH = D // 2

NC = 2
NS = 16
CHUNK = 128
NCHC = E // CHUNK
NCHM = NCHC // NS
XTRA = NCHC - NS * NCHM
NCH = NCHM + 1
R = 10112
RPT = R // NS
BLK = 2000
GRID = N // BLK
CW = 8
NB = 4


def _sc_scatter(with_counts):
    out_type = [jax.ShapeDtypeStruct((NC, R, DH), jnp.float32)]
    if with_counts:
        out_type.append(jax.ShapeDtypeStruct((NC, R, CW), jnp.float32))
    scratch = [
        pltpu.VMEM((NCH, CHUNK), jnp.int32),
        pltpu.VMEM((NCH, CHUNK), jnp.int32),
        pltpu.VMEM((NB, CHUNK, DH), jnp.float32),
        pltpu.VMEM((CHUNK, CW), jnp.float32),
        pltpu.VMEM_SHARED((R, DH), jnp.float32),
        pltpu.VMEM_SHARED((R, CW), jnp.float32),
    ] + [pltpu.SemaphoreType.DMA] * NB

    mesh = plsc.VectorSubcoreMesh(core_axis_name="c", subcore_axis_name="s")

    @functools.partial(
        pl.kernel, out_type=tuple(out_type), mesh=mesh,
        scratch_types=scratch,
        compiler_params=pltpu.CompilerParams(use_tc_tiling_on_sc=False))
    def k(xr_hbm, src_hbm, dst_hbm, zeros_hbm, zeros_c_hbm, ones_hbm,
          part_hbm, *rest):
        if with_counts:
            cnt_hbm = rest[0]
            rest = rest[1:]
        src_v, dst_v, rows_v, ones_v, acc, cacc = rest[:6]
        sems_g = rest[6:]
        c = lax.axis_index("c")
        s = lax.axis_index("s")

        pltpu.sync_copy(zeros_hbm, acc.at[pl.ds(s * RPT, RPT)])
        if with_counts:
            pltpu.sync_copy(zeros_c_hbm, cacc.at[pl.ds(s * RPT, RPT)])
            pltpu.sync_copy(ones_hbm, ones_v)
        base = s * NCHM
        pltpu.sync_copy(src_hbm.at[c, pl.ds(base, NCHM)],
                        src_v.at[pl.ds(0, NCHM)])
        pltpu.sync_copy(dst_hbm.at[pl.ds(base, NCHM)],
                        dst_v.at[pl.ds(0, NCHM)])

        @pl.when(s < XTRA)
        def _():
            pltpu.sync_copy(src_hbm.at[c, pl.ds(NS * NCHM + s, 1)],
                            src_v.at[pl.ds(NCHM, 1)])
            pltpu.sync_copy(dst_hbm.at[pl.ds(NS * NCHM + s, 1)],
                            dst_v.at[pl.ds(NCHM, 1)])

        plsc.subcore_barrier()

        for b in range(NB):
            pltpu.async_copy(xr_hbm.at[src_v.at[b]], rows_v.at[b], sems_g[b])

        @pl.loop(0, NCHM, step=NB)
        def chunk_loop(j):
            for b in range(NB):
                jj = j + b
                pltpu.make_async_copy(
                    xr_hbm.at[src_v.at[jj]], rows_v.at[b], sems_g[b]).wait()
                pltpu.sync_copy(rows_v.at[b], acc.at[dst_v.at[jj]], add=True)
                if with_counts:
                    @pl.when(c == b % NC)
                    def _():
                        pltpu.sync_copy(ones_v, cacc.at[dst_v.at[jj]],
                                        add=True)
                nxt = jj + NB

                @pl.when(nxt < NCHM)
                def _():
                    pltpu.async_copy(xr_hbm.at[src_v.at[nxt]], rows_v.at[b],
                                     sems_g[b])

        @pl.when(s < XTRA)
        def _():
            pltpu.async_copy(xr_hbm.at[src_v.at[NCHM]], rows_v.at[0],
                             sems_g[0]).wait()
            pltpu.sync_copy(rows_v.at[0], acc.at[dst_v.at[NCHM]], add=True)
            if with_counts:
                @pl.when(c == s % NC)
                def _():
                    pltpu.sync_copy(ones_v, cacc.at[dst_v.at[NCHM]],
                                    add=True)

        plsc.subcore_barrier()
        rows = pl.ds(s * RPT, RPT)
        pltpu.sync_copy(acc.at[rows], part_hbm.at[c, rows])
        if with_counts:
            pltpu.sync_copy(cacc.at[rows], cnt_hbm.at[c, rows])

    return k


_sc_scatter_l1 = _sc_scatter(True)
_sc_scatter_l2 = _sc_scatter(False)


def _dotT(a, w):
    return lax.dot_general(a, w, (((1,), (1,)), ((), ())),
                           preferred_element_type=jnp.float32)


def _tc_root(x, Wr, b):
    def body(x_ref, wr_ref, b_ref, y_ref):
        y_ref[...] = _dotT(x_ref[...], wr_ref[...]) + b_ref[...]

    return pl.pallas_call(
        body,
        grid=(GRID,),
        in_specs=[
            pl.BlockSpec((BLK, D), lambda i: (i, 0)),
            pl.BlockSpec((D, D), lambda i: (0, 0)),
            pl.BlockSpec((1, D), lambda i: (0, 0)),
        ],
        out_specs=pl.BlockSpec((BLK, D), lambda i: (i, 0)),
        out_shape=jax.ShapeDtypeStruct((N, D), jnp.float32),
    )(x, Wr, b.reshape(1, D))


def _tc_mid(p, cnt, y1, Wl, Wr, b):
    def body(p_ref, c_ref, y1_ref, wl_ref, wr_ref, b_ref, h_ref, y_ref):
        agg = jnp.concatenate([p_ref[0], p_ref[1]], axis=1)
        cn = c_ref[0, :, :1] + c_ref[1, :, :1]
        mean = agg * (1.0 / jnp.maximum(cn, 1.0))
        h = jnp.maximum(_dotT(mean, wl_ref[...]) + y1_ref[...], 0.0)
        h_ref[...] = h
        y_ref[...] = _dotT(h, wr_ref[...]) + b_ref[...]

    h, y2 = pl.pallas_call(
        body,
        grid=(GRID,),
        in_specs=[
            pl.BlockSpec((NC, BLK, DH), lambda i: (0, i, 0)),
            pl.BlockSpec((NC, BLK, CW), lambda i: (0, i, 0)),
            pl.BlockSpec((BLK, D), lambda i: (i, 0)),
            pl.BlockSpec((D, D), lambda i: (0, 0)),
            pl.BlockSpec((D, D), lambda i: (0, 0)),
            pl.BlockSpec((1, D), lambda i: (0, 0)),
        ],
        out_specs=[
            pl.BlockSpec((BLK, D), lambda i: (i, 0)),
            pl.BlockSpec((BLK, D), lambda i: (i, 0)),
        ],
        out_shape=[
            jax.ShapeDtypeStruct((N, D), jnp.float32),
            jax.ShapeDtypeStruct((N, D), jnp.float32),
        ],
    )(p, cnt, y1, Wl, Wr, b.reshape(1, D))
    return h, y2


def _tc_out(q, cnt, y2, Wl):
    def body(q_ref, c_ref, y2_ref, wl_ref, o_ref):
        agg = jnp.concatenate([q_ref[0], q_ref[1]], axis=1)
        cn = c_ref[0, :, :1] + c_ref[1, :, :1]
        mean = agg * (1.0 / jnp.maximum(cn, 1.0))
        o_ref[...] = _dotT(mean, wl_ref[...]) + y2_ref[...]

    return pl.pallas_call(
        body,
        grid=(GRID,),
        in_specs=[
            pl.BlockSpec((NC, BLK, DH), lambda i: (0, i, 0)),
            pl.BlockSpec((NC, BLK, CW), lambda i: (0, i, 0)),
            pl.BlockSpec((BLK, D), lambda i: (i, 0)),
            pl.BlockSpec((D, D), lambda i: (0, 0)),
        ],
        out_specs=pl.BlockSpec((BLK, D), lambda i: (i, 0)),
        out_shape=jax.ShapeDtypeStruct((N, D), jnp.float32),
    )(q, cnt, y2, Wl)


def kernel(x, edge_index, W1l, b1l, W1r, W2l, b2l, W2r):
    src_r = edge_index[0].reshape(NCHC, CHUNK).astype(jnp.int32)
    dst_r = edge_index[1].reshape(NCHC, CHUNK).astype(jnp.int32)
    src2 = jnp.stack([src_r * 2, src_r * 2 + 1])
    zeros = jnp.zeros((RPT, DH), jnp.float32)
    zeros_c = jnp.zeros((RPT, CW), jnp.float32)
    ones = jnp.ones((CHUNK, CW), jnp.float32)

    xr = x.reshape(2 * N, DH)
    p, cnt = _sc_scatter_l1(xr, src2, dst_r, zeros, zeros_c, ones)
    y1 = _tc_root(x, W1r, b1l)
    h, y2 = _tc_mid(p, cnt, y1, W1l, W2r, b2l)
    hr = h.reshape(2 * N, DH)
    (q,) = _sc_scatter_l2(hr, src2, dst_r, zeros, zeros_c, ones)
    return _tc_out(q, cnt, y2, W2l)


# speedup vs baseline: 1.1822x; 1.0928x over previous
import functools

import jax
import jax.numpy as jnp
from jax import lax
from jax.experimental import pallas as pl
from jax.experimental.pallas import tpu as pltpu, tpu_sc as plsc

N = 10000
E = 320000
D = 128
DH = D // 2

NC = 2
NS = 16
CHUNK = 128
NCHC = E // CHUNK
NCHM = NCHC // NS
XTRA = NCHC - NS * NCHM
NCH = NCHM + 1
R = 10112
RPT = R // NS
BLK = 2000
GRID = N // BLK
CW = 8
NB = 4


def _sc_scatter(with_counts):
    out_type = [jax.ShapeDtypeStruct((R, D), jnp.float32)]
    if with_counts:
        out_type.append(jax.ShapeDtypeStruct((NC, R, CW), jnp.float32))
    scratch = [
        pltpu.VMEM((NCH, CHUNK), jnp.int32),
        pltpu.VMEM((NCH, CHUNK), jnp.int32),
        pltpu.VMEM((NB, CHUNK, DH), jnp.float32),
        pltpu.VMEM((CHUNK, CW), jnp.float32),
        pltpu.VMEM_SHARED((R, DH), jnp.float32),
        pltpu.VMEM_SHARED((R, CW), jnp.float32),
    ] + [pltpu.SemaphoreType.DMA] * NB

    mesh = plsc.VectorSubcoreMesh(core_axis_name="c", subcore_axis_name="s")

    @functools.partial(
        pl.kernel, out_type=tuple(out_type), mesh=mesh,
        scratch_types=scratch,
        compiler_params=pltpu.CompilerParams(use_tc_tiling_on_sc=False))
    def k(xr_hbm, src_hbm, dst_hbm, zeros_hbm, zeros_c_hbm, ones_hbm,
          part_hbm, *rest):
        if with_counts:
            cnt_hbm = rest[0]
            rest = rest[1:]
        src_v, dst_v, rows_v, ones_v, acc, cacc = rest[:6]
        sems_g = rest[6:]
        c = lax.axis_index("c")
        s = lax.axis_index("s")

        pltpu.sync_copy(zeros_hbm, acc.at[pl.ds(s * RPT, RPT)])
        if with_counts:
            pltpu.sync_copy(zeros_c_hbm, cacc.at[pl.ds(s * RPT, RPT)])
            pltpu.sync_copy(ones_hbm, ones_v)
        base = s * NCHM
        pltpu.sync_copy(src_hbm.at[c, pl.ds(base, NCHM)],
                        src_v.at[pl.ds(0, NCHM)])
        pltpu.sync_copy(dst_hbm.at[pl.ds(base, NCHM)],
                        dst_v.at[pl.ds(0, NCHM)])

        @pl.when(s < XTRA)
        def _():
            pltpu.sync_copy(src_hbm.at[c, pl.ds(NS * NCHM + s, 1)],
                            src_v.at[pl.ds(NCHM, 1)])
            pltpu.sync_copy(dst_hbm.at[pl.ds(NS * NCHM + s, 1)],
                            dst_v.at[pl.ds(NCHM, 1)])

        plsc.subcore_barrier()

        for b in range(NB):
            pltpu.async_copy(xr_hbm.at[src_v.at[b]], rows_v.at[b], sems_g[b])

        @pl.loop(0, NCHM, step=NB)
        def chunk_loop(j):
            for b in range(NB):
                jj = j + b
                pltpu.make_async_copy(
                    xr_hbm.at[src_v.at[jj]], rows_v.at[b], sems_g[b]).wait()
                pltpu.sync_copy(rows_v.at[b], acc.at[dst_v.at[jj]], add=True)
                if with_counts:
                    @pl.when(c == b % NC)
                    def _():
                        pltpu.sync_copy(ones_v, cacc.at[dst_v.at[jj]],
                                        add=True)
                nxt = jj + NB

                @pl.when(nxt < NCHM)
                def _():
                    pltpu.async_copy(xr_hbm.at[src_v.at[nxt]], rows_v.at[b],
                                     sems_g[b])

        @pl.when(s < XTRA)
        def _():
            pltpu.async_copy(xr_hbm.at[src_v.at[NCHM]], rows_v.at[0],
                             sems_g[0]).wait()
            pltpu.sync_copy(rows_v.at[0], acc.at[dst_v.at[NCHM]], add=True)
            if with_counts:
                @pl.when(c == s % NC)
                def _():
                    pltpu.sync_copy(ones_v, cacc.at[dst_v.at[NCHM]],
                                    add=True)

        plsc.subcore_barrier()
        rows = pl.ds(s * RPT, RPT)
        pltpu.sync_copy(acc.at[rows],
                        part_hbm.at[rows, pl.ds(c * DH, DH)])
        if with_counts:
            pltpu.sync_copy(cacc.at[rows], cnt_hbm.at[c, rows])

    return k


_sc_scatter_l1 = _sc_scatter(True)
_sc_scatter_l2 = _sc_scatter(False)


def _dotT(a, w):
    return lax.dot_general(a, w, (((1,), (1,)), ((), ())),
                           preferred_element_type=jnp.float32)


def _tc_root(x, Wr, b):
    def body(x_ref, wr_ref, b_ref, y_ref):
        y_ref[...] = _dotT(x_ref[...], wr_ref[...]) + b_ref[...]

    return pl.pallas_call(
        body,
        grid=(GRID,),
        in_specs=[
            pl.BlockSpec((BLK, D), lambda i: (i, 0)),
            pl.BlockSpec((D, D), lambda i: (0, 0)),
            pl.BlockSpec((1, D), lambda i: (0, 0)),
        ],
        out_specs=pl.BlockSpec((BLK, D), lambda i: (i, 0)),
        out_shape=jax.ShapeDtypeStruct((N, D), jnp.float32),
    )(x, Wr, b.reshape(1, D))


def _tc_mid(p, cnt, y1, Wl, Wr, b):
    def body(p_ref, c_ref, y1_ref, wl_ref, wr_ref, b_ref, h_ref, y_ref):
        cn = c_ref[0, :, :1] + c_ref[1, :, :1]
        mean = p_ref[...] * (1.0 / jnp.maximum(cn, 1.0))
        h = jnp.maximum(_dotT(mean, wl_ref[...]) + y1_ref[...], 0.0)
        h_ref[...] = h
        y_ref[...] = _dotT(h, wr_ref[...]) + b_ref[...]

    h, y2 = pl.pallas_call(
        body,
        grid=(GRID,),
        in_specs=[
            pl.BlockSpec((BLK, D), lambda i: (i, 0)),
            pl.BlockSpec((NC, BLK, CW), lambda i: (0, i, 0)),
            pl.BlockSpec((BLK, D), lambda i: (i, 0)),
            pl.BlockSpec((D, D), lambda i: (0, 0)),
            pl.BlockSpec((D, D), lambda i: (0, 0)),
            pl.BlockSpec((1, D), lambda i: (0, 0)),
        ],
        out_specs=[
            pl.BlockSpec((BLK, D), lambda i: (i, 0)),
            pl.BlockSpec((BLK, D), lambda i: (i, 0)),
        ],
        out_shape=[
            jax.ShapeDtypeStruct((N, D), jnp.float32),
            jax.ShapeDtypeStruct((N, D), jnp.float32),
        ],
    )(p, cnt, y1, Wl, Wr, b.reshape(1, D))
    return h, y2


def _tc_out(q, cnt, y2, Wl):
    def body(q_ref, c_ref, y2_ref, wl_ref, o_ref):
        cn = c_ref[0, :, :1] + c_ref[1, :, :1]
        mean = q_ref[...] * (1.0 / jnp.maximum(cn, 1.0))
        o_ref[...] = _dotT(mean, wl_ref[...]) + y2_ref[...]

    return pl.pallas_call(
        body,
        grid=(GRID,),
        in_specs=[
            pl.BlockSpec((BLK, D), lambda i: (i, 0)),
            pl.BlockSpec((NC, BLK, CW), lambda i: (0, i, 0)),
            pl.BlockSpec((BLK, D), lambda i: (i, 0)),
            pl.BlockSpec((D, D), lambda i: (0, 0)),
        ],
        out_specs=pl.BlockSpec((BLK, D), lambda i: (i, 0)),
        out_shape=jax.ShapeDtypeStruct((N, D), jnp.float32),
    )(q, cnt, y2, Wl)


def kernel(x, edge_index, W1l, b1l, W1r, W2l, b2l, W2r):
    src_r = edge_index[0].reshape(NCHC, CHUNK).astype(jnp.int32)
    dst_r = edge_index[1].reshape(NCHC, CHUNK).astype(jnp.int32)
    src2 = jnp.stack([src_r * 2, src_r * 2 + 1])
    zeros = jnp.zeros((RPT, DH), jnp.float32)
    zeros_c = jnp.zeros((RPT, CW), jnp.float32)
    ones = jnp.ones((CHUNK, CW), jnp.float32)

    xr = x.reshape(2 * N, DH)
    p, cnt = _sc_scatter_l1(xr, src2, dst_r, zeros, zeros_c, ones)
    y1 = _tc_root(x, W1r, b1l)
    h, y2 = _tc_mid(p, cnt, y1, W1l, W2r, b2l)
    hr = h.reshape(2 * N, DH)
    (q,) = _sc_scatter_l2(hr, src2, dst_r, zeros, zeros_c, ones)
    return _tc_out(q, cnt, y2, W2l)
